# baseline (speedup 1.0000x reference)
import functools

import jax
import jax.numpy as jnp
from jax import lax
from jax.experimental import pallas as pl
from jax.experimental.pallas import tpu as pltpu
from jax.experimental.pallas import tpu_sc as plsc

INPUT_DIM = 1000
OUTPUT_DIM = 64
ROW = OUTPUT_DIM * OUTPUT_DIM
BATCH = 4096

NUM_CORES = 2
NUM_SUBCORES = 16
NUM_WORKERS = NUM_CORES * NUM_SUBCORES
B_PER_W = BATCH // NUM_WORKERS
CHUNK = 8
NCHUNK = B_PER_W // CHUNK
NBUF = 3


def _build(batch):
  b_per_w = batch // NUM_WORKERS
  nchunk = b_per_w // CHUNK
  mesh = plsc.VectorSubcoreMesh(core_axis_name="c", subcore_axis_name="s")

  @functools.partial(
      pl.kernel,
      mesh=mesh,
      out_type=jax.ShapeDtypeStruct((batch, ROW), jnp.float32),
      scratch_types=(
          [pltpu.VMEM((b_per_w,), jnp.int32)]
          + [pltpu.VMEM((CHUNK, ROW), jnp.float32)] * NBUF
          + [pltpu.SemaphoreType.DMA] * (2 * NBUF)
      ),
  )
  def gather_kernel(idx_hbm, table_hbm, out_hbm, idx_v, *rest):
    bufs = rest[:NBUF]
    gsems = rest[NBUF:2 * NBUF]
    ssems = rest[2 * NBUF:]
    wid = lax.axis_index("s") * NUM_CORES + lax.axis_index("c")
    base = wid * b_per_w
    pltpu.sync_copy(idx_hbm.at[pl.ds(base, b_per_w)], idx_v)

    def gather(g, b):
      return pltpu.async_copy(
          table_hbm.at[idx_v.at[pl.ds(g * CHUNK, CHUNK)]], bufs[b], gsems[b])

    def scatter(g, b):
      return pltpu.async_copy(
          bufs[b], out_hbm.at[pl.ds(base + g * CHUNK, CHUNK)], ssems[b])

    gd = [None] * nchunk
    sd = [None] * nchunk
    gd[0] = gather(0, 0)
    gd[1] = gather(1, 1)
    for g in range(nchunk):
      b = g % NBUF
      gd[g].wait()
      sd[g] = scatter(g, b)
      if g + 2 < nchunk:
        if g >= 1:
          sd[g - 1].wait()
        gd[g + 2] = gather(g + 2, (g + 2) % NBUF)
    sd[nchunk - 3].wait()
    sd[nchunk - 2].wait()
    sd[nchunk - 1].wait()

  return gather_kernel


NPART = 4
PART = BATCH // NPART

_gather_part = _build(PART)

_OUT_SHAPE = jax.ShapeDtypeStruct((BATCH, OUTPUT_DIM, OUTPUT_DIM),
                                  jnp.float32)


def _alloc_body(out_ref):
  del out_ref


_alloc_out = pl.pallas_call(
    _alloc_body,
    out_specs=pl.BlockSpec(memory_space=pl.ANY),
    out_shape=_OUT_SHAPE,
)


def kernel(inputs, embeddings):
  table = embeddings.reshape(INPUT_DIM, ROW)
  out = _alloc_out()
  for k in range(NPART):
    flat = _gather_part(
        lax.slice(inputs, (k * PART,), ((k + 1) * PART,)), table)
    out = lax.dynamic_update_slice(
        out, flat.reshape(PART, OUTPUT_DIM, OUTPUT_DIM), (k * PART, 0, 0))
  return out


# speedup vs baseline: 2.3177x; 2.3177x over previous
import functools

import jax
import jax.numpy as jnp
from jax import lax
from jax.experimental import pallas as pl
from jax.experimental.pallas import tpu as pltpu
from jax.experimental.pallas import tpu_sc as plsc

INPUT_DIM = 1000
OUTPUT_DIM = 64
ROW = OUTPUT_DIM * OUTPUT_DIM
BATCH = 4096

NUM_CORES = 2
NUM_SUBCORES = 16
NUM_WORKERS = NUM_CORES * NUM_SUBCORES
B_PER_W = BATCH // NUM_WORKERS
CHUNK = 8
NCHUNK = B_PER_W // CHUNK
NBUF = 3


def _build():
  mesh = plsc.VectorSubcoreMesh(core_axis_name="c", subcore_axis_name="s")

  @functools.partial(
      pl.kernel,
      mesh=mesh,
      out_type=jax.ShapeDtypeStruct((BATCH, ROW), jnp.float32),
      scratch_types=(
          [pltpu.VMEM((B_PER_W,), jnp.int32)]
          + [pltpu.VMEM((CHUNK, ROW), jnp.float32)] * NBUF
          + [pltpu.SemaphoreType.DMA] * (2 * NBUF)
      ),
  )
  def gather_kernel(idx_hbm, table_hbm, out_hbm, idx_v, *rest):
    bufs = rest[:NBUF]
    gsems = rest[NBUF:2 * NBUF]
    ssems = rest[2 * NBUF:]
    wid = lax.axis_index("s") * NUM_CORES + lax.axis_index("c")
    base = wid * B_PER_W
    pltpu.sync_copy(idx_hbm.at[pl.ds(base, B_PER_W)], idx_v)

    def gather(g, b):
      return pltpu.async_copy(
          table_hbm.at[idx_v.at[pl.ds(g * CHUNK, CHUNK)]], bufs[b], gsems[b])

    def scatter(g, b):
      return pltpu.async_copy(
          bufs[b], out_hbm.at[pl.ds(base + g * CHUNK, CHUNK)], ssems[b])

    gd = [None] * NCHUNK
    sd = [None] * NCHUNK
    gd[0] = gather(0, 0)
    gd[1] = gather(1, 1)
    for g in range(NCHUNK):
      b = g % NBUF
      gd[g].wait()
      sd[g] = scatter(g, b)
      if g + 2 < NCHUNK:
        if g >= 1:
          sd[g - 1].wait()
        gd[g + 2] = gather(g + 2, (g + 2) % NBUF)
    sd[NCHUNK - 3].wait()
    sd[NCHUNK - 2].wait()
    sd[NCHUNK - 1].wait()

  return gather_kernel


_gather = _build()


def kernel(inputs, embeddings):
  table = embeddings.reshape(INPUT_DIM, ROW)
  out = _gather(inputs, table)
  return out.reshape(BATCH, OUTPUT_DIM, OUTPUT_DIM)
